# baseline (speedup 1.0000x reference)
import functools

import jax
import jax.numpy as jnp
from jax import lax
from jax.experimental import pallas as pl
from jax.experimental.pallas import tpu as pltpu
from jax.experimental.pallas import tpu_sc as plsc

T = 2048
D = 1024
F = 2048
E = 8
K = 2
TK = T * K

BLK = 128
NB = TK // BLK + E - 1
NPAD = NB * BLK


def _routing_metadata(selected_experts):
    se = selected_experts.reshape(TK).astype(jnp.int32)
    onehot = (se[:, None] == jnp.arange(E, dtype=jnp.int32)[None, :]).astype(jnp.int32)
    ccum = jnp.cumsum(onehot, axis=0)
    counts = ccum[-1]
    rank = jnp.sum(onehot * ccum, axis=1) - 1
    padded = ((counts + BLK - 1) // BLK) * BLK
    bend = jnp.cumsum(padded)
    astart = bend - padded
    p = jnp.sum(onehot * astart[None, :], axis=1) + rank
    blk_base = jnp.arange(NB, dtype=jnp.int32) * BLK
    block_expert = jnp.sum((blk_base[:, None] >= bend[None, :]).astype(jnp.int32), axis=1)
    block_expert = jnp.minimum(block_expert, E - 1)
    nact = (bend[-1] + BLK - 1) // BLK
    return p, block_expert, nact.reshape(1).astype(jnp.int32)


NW = 32
TPW = T // NW
WCOL = 128


def _ffn_block_kernel(be_ref, nact_ref, x_ref, g_ref, u_ref, d_ref, w_ref, y_ref):
    @pl.when(pl.program_id(0) < nact_ref[0])
    def _():
        x = x_ref[...].astype(jnp.bfloat16)
        g = jnp.dot(x, g_ref[0], preferred_element_type=jnp.float32)
        u = jnp.dot(x, u_ref[0], preferred_element_type=jnp.float32)
        h = (g * jax.nn.sigmoid(g) * u).astype(jnp.bfloat16)
        y = jnp.dot(h, d_ref[0], preferred_element_type=jnp.float32)
        y_ref[...] = y * w_ref[:, 0:1]


def _grouped_ffn(block_expert, nact, x_sorted, gate_proj, up_proj, down_proj, w_sorted):
    grid_spec = pltpu.PrefetchScalarGridSpec(
        num_scalar_prefetch=2,
        grid=(NB,),
        in_specs=[
            pl.BlockSpec((BLK, D), lambda i, be, na: (i, 0)),
            pl.BlockSpec((1, D, F), lambda i, be, na: (be[i], 0, 0)),
            pl.BlockSpec((1, D, F), lambda i, be, na: (be[i], 0, 0)),
            pl.BlockSpec((1, F, D), lambda i, be, na: (be[i], 0, 0)),
            pl.BlockSpec((BLK, WCOL), lambda i, be, na: (i, 0)),
        ],
        out_specs=pl.BlockSpec((BLK, D), lambda i, be, na: (i, 0)),
    )
    return pl.pallas_call(
        _ffn_block_kernel,
        grid_spec=grid_spec,
        out_shape=jax.ShapeDtypeStruct((NPAD, D), jnp.float32),
    )(block_expert, nact, x_sorted, gate_proj, up_proj, down_proj, w_sorted)


def _sc_dispatch(hidden_states, p0, p1, rwb0, rwb1):
    mesh = plsc.VectorSubcoreMesh(core_axis_name="c", subcore_axis_name="s")

    @functools.partial(
        pl.kernel,
        mesh=mesh,
        out_type=(
            jax.ShapeDtypeStruct((NPAD, D), jnp.float32),
            jax.ShapeDtypeStruct((NPAD, WCOL), jnp.float32),
        ),
        scratch_types=[
            pltpu.VMEM((TPW,), jnp.int32),
            pltpu.VMEM((TPW,), jnp.int32),
            pltpu.VMEM((TPW, D), jnp.float32),
            pltpu.VMEM((TPW, WCOL), jnp.float32),
            pltpu.SemaphoreType.DMA,
        ],
    )
    def k(hid_hbm, p0_hbm, p1_hbm, w0_hbm, w1_hbm, x_hbm, ws_hbm,
          i0_v, i1_v, rows_v, w_v, sem):
        wid = lax.axis_index("s") * 2 + lax.axis_index("c")
        base = wid * TPW
        pltpu.sync_copy(p0_hbm.at[pl.ds(base, TPW)], i0_v)
        pltpu.sync_copy(p1_hbm.at[pl.ds(base, TPW)], i1_v)
        pltpu.sync_copy(hid_hbm.at[pl.ds(base, TPW)], rows_v)
        pltpu.async_copy(rows_v, x_hbm.at[i0_v], sem).wait()
        pltpu.async_copy(rows_v, x_hbm.at[i1_v], sem).wait()
        pltpu.sync_copy(w0_hbm.at[pl.ds(base, TPW)], w_v)
        pltpu.async_copy(w_v, ws_hbm.at[i0_v], sem).wait()
        pltpu.sync_copy(w1_hbm.at[pl.ds(base, TPW)], w_v)
        pltpu.async_copy(w_v, ws_hbm.at[i1_v], sem).wait()

    return k(hidden_states, p0, p1, rwb0, rwb1)


CH = 32


def _sc_combine(y_sorted, p0, p1):
    mesh = plsc.VectorSubcoreMesh(core_axis_name="c", subcore_axis_name="s")

    @functools.partial(
        pl.kernel,
        mesh=mesh,
        out_type=jax.ShapeDtypeStruct((T, D), jnp.float32),
        scratch_types=[
            pltpu.VMEM((TPW,), jnp.int32),
            pltpu.VMEM((TPW,), jnp.int32),
            pltpu.VMEM((CH, D), jnp.float32),
            pltpu.VMEM((CH, D), jnp.float32),
            pltpu.SemaphoreType.DMA,
        ],
    )
    def k(y_hbm, p0_hbm, p1_hbm, out_hbm, i0_v, i1_v, y0_v, y1_v, sem):
        wid = lax.axis_index("s") * 2 + lax.axis_index("c")
        base = wid * TPW
        pltpu.sync_copy(p0_hbm.at[pl.ds(base, TPW)], i0_v)
        pltpu.sync_copy(p1_hbm.at[pl.ds(base, TPW)], i1_v)
        for ch in range(TPW // CH):
            pltpu.async_copy(y_hbm.at[i0_v.at[pl.ds(ch * CH, CH)]], y0_v, sem).wait()
            pltpu.async_copy(y_hbm.at[i1_v.at[pl.ds(ch * CH, CH)]], y1_v, sem).wait()

            def body(t, _):
                for j in range(D // 16):
                    sl = pl.ds(j * 16, 16)
                    y0_v[t, sl] = y0_v[t, sl] + y1_v[t, sl]
                return 0

            lax.fori_loop(0, CH, body, 0)
            pltpu.sync_copy(y0_v, out_hbm.at[pl.ds(base + ch * CH, CH)])

    return k(y_sorted, p0, p1)


def kernel(hidden_states, routing_weights, selected_experts, gate_proj, up_proj, down_proj):
    p, block_expert, nact = _routing_metadata(selected_experts)
    pk = p.reshape(T, K)
    p0 = pk[:, 0]
    p1 = pk[:, 1]
    rwb0 = jnp.broadcast_to(routing_weights[:, 0:1], (T, WCOL))
    rwb1 = jnp.broadcast_to(routing_weights[:, 1:2], (T, WCOL))

    x_sorted, w_sorted = _sc_dispatch(hidden_states, p0, p1, rwb0, rwb1)

    y_sorted = _grouped_ffn(block_expert, nact, x_sorted,
                            gate_proj.astype(jnp.bfloat16),
                            up_proj.astype(jnp.bfloat16),
                            down_proj.astype(jnp.bfloat16), w_sorted)

    return _sc_combine(y_sorted, p0, p1)


# speedup vs baseline: 1.3288x; 1.3288x over previous
import functools

import jax
import jax.numpy as jnp
from jax import lax
from jax.experimental import pallas as pl
from jax.experimental.pallas import tpu as pltpu
from jax.experimental.pallas import tpu_sc as plsc

T = 2048
D = 1024
F = 2048
E = 8
K = 2
TK = T * K

BLK = 128
NB = TK // BLK + E - 1
NPAD = NB * BLK


def _routing_metadata(selected_experts):
    se = selected_experts.reshape(TK).astype(jnp.int32)
    onehot = (se[:, None] == jnp.arange(E, dtype=jnp.int32)[None, :]).astype(jnp.float32)
    SCB = 128
    NSB = TK // SCB
    ohb = onehot.reshape(NSB, SCB, E)
    tri = jnp.tril(jnp.ones((SCB, SCB), jnp.float32))
    inner = jax.lax.dot_general(tri, ohb, (((1,), (1,)), ((), ())),
                                preferred_element_type=jnp.float32)
    bsum = inner[SCB - 1]
    trib = jnp.tril(jnp.ones((NSB, NSB), jnp.float32), k=-1)
    boff = jnp.dot(trib, bsum, preferred_element_type=jnp.float32)
    ccum = (jnp.transpose(inner, (1, 0, 2)) + boff[:, None, :]).reshape(TK, E)
    counts = (ccum[-1] + 0.5).astype(jnp.int32)
    rank = (jnp.sum(onehot * ccum, axis=1) + 0.5).astype(jnp.int32) - 1
    padded = ((counts + BLK - 1) // BLK) * BLK
    tril8 = jnp.tril(jnp.ones((E, E), jnp.int32))
    bend = jnp.sum(tril8 * padded[None, :], axis=1)
    astart = bend - padded
    p = (jnp.sum(onehot * astart[None, :].astype(jnp.float32), axis=1) + 0.5).astype(jnp.int32) + rank
    blk_base = jnp.arange(NB, dtype=jnp.int32) * BLK
    block_expert = jnp.sum((blk_base[:, None] >= bend[None, :]).astype(jnp.int32), axis=1)
    block_expert = jnp.minimum(block_expert, E - 1)
    nact = (bend[-1] + BLK - 1) // BLK
    return p, block_expert, nact.reshape(1).astype(jnp.int32)


NW = 32
TPW = T // NW
WCOL = 128


def _ffn_block_kernel(be_ref, nact_ref, x_ref, g_ref, u_ref, d_ref, w_ref, y_ref):
    @pl.when(pl.program_id(0) < nact_ref[0])
    def _():
        x = x_ref[...]
        g = jnp.dot(x, g_ref[0], preferred_element_type=jnp.float32)
        u = jnp.dot(x, u_ref[0], preferred_element_type=jnp.float32)
        h = g * jax.nn.sigmoid(g) * u
        y = jnp.dot(h, d_ref[0], preferred_element_type=jnp.float32)
        y_ref[...] = y * w_ref[:, 0:1]


def _grouped_ffn(block_expert, nact, x_sorted, gate_proj, up_proj, down_proj, w_sorted):
    grid_spec = pltpu.PrefetchScalarGridSpec(
        num_scalar_prefetch=2,
        grid=(NB,),
        in_specs=[
            pl.BlockSpec((BLK, D), lambda i, be, na: (i, 0)),
            pl.BlockSpec((1, D, F), lambda i, be, na: (be[i], 0, 0)),
            pl.BlockSpec((1, D, F), lambda i, be, na: (be[i], 0, 0)),
            pl.BlockSpec((1, F, D), lambda i, be, na: (be[i], 0, 0)),
            pl.BlockSpec((BLK, WCOL), lambda i, be, na: (i, 0)),
        ],
        out_specs=pl.BlockSpec((BLK, D), lambda i, be, na: (i, 0)),
    )
    return pl.pallas_call(
        _ffn_block_kernel,
        grid_spec=grid_spec,
        out_shape=jax.ShapeDtypeStruct((NPAD, D), jnp.float32),
    )(block_expert, nact, x_sorted, gate_proj, up_proj, down_proj, w_sorted)


def _sc_dispatch(hidden_states, p0, p1, rwb0, rwb1):
    mesh = plsc.VectorSubcoreMesh(core_axis_name="c", subcore_axis_name="s")

    @functools.partial(
        pl.kernel,
        mesh=mesh,
        out_type=(
            jax.ShapeDtypeStruct((NPAD, D), jnp.float32),
            jax.ShapeDtypeStruct((NPAD, WCOL), jnp.float32),
        ),
        scratch_types=[
            pltpu.VMEM((TPW,), jnp.int32),
            pltpu.VMEM((TPW,), jnp.int32),
            pltpu.VMEM((TPW, D), jnp.float32),
            pltpu.VMEM((TPW, WCOL), jnp.float32),
            pltpu.SemaphoreType.DMA,
        ],
    )
    def k(hid_hbm, p0_hbm, p1_hbm, w0_hbm, w1_hbm, x_hbm, ws_hbm,
          i0_v, i1_v, rows_v, w_v, sem):
        wid = lax.axis_index("s") * 2 + lax.axis_index("c")
        base = wid * TPW
        pltpu.sync_copy(p0_hbm.at[pl.ds(base, TPW)], i0_v)
        pltpu.sync_copy(p1_hbm.at[pl.ds(base, TPW)], i1_v)
        pltpu.sync_copy(hid_hbm.at[pl.ds(base, TPW)], rows_v)
        pltpu.async_copy(rows_v, x_hbm.at[i0_v], sem).wait()
        pltpu.async_copy(rows_v, x_hbm.at[i1_v], sem).wait()
        pltpu.sync_copy(w0_hbm.at[pl.ds(base, TPW)], w_v)
        pltpu.async_copy(w_v, ws_hbm.at[i0_v], sem).wait()
        pltpu.sync_copy(w1_hbm.at[pl.ds(base, TPW)], w_v)
        pltpu.async_copy(w_v, ws_hbm.at[i1_v], sem).wait()

    return k(hidden_states, p0, p1, rwb0, rwb1)


CH = 32


def _sc_combine(y_sorted, p0, p1):
    mesh = plsc.VectorSubcoreMesh(core_axis_name="c", subcore_axis_name="s")

    @functools.partial(
        pl.kernel,
        mesh=mesh,
        out_type=jax.ShapeDtypeStruct((T, D), jnp.float32),
        scratch_types=[
            pltpu.VMEM((TPW,), jnp.int32),
            pltpu.VMEM((TPW,), jnp.int32),
            pltpu.VMEM((CH, D), jnp.float32),
            pltpu.VMEM((CH, D), jnp.float32),
            pltpu.SemaphoreType.DMA,
        ],
    )
    def k(y_hbm, p0_hbm, p1_hbm, out_hbm, i0_v, i1_v, y0_v, y1_v, sem):
        wid = lax.axis_index("s") * 2 + lax.axis_index("c")
        base = wid * TPW
        pltpu.sync_copy(p0_hbm.at[pl.ds(base, TPW)], i0_v)
        pltpu.sync_copy(p1_hbm.at[pl.ds(base, TPW)], i1_v)
        for ch in range(TPW // CH):
            pltpu.async_copy(y_hbm.at[i0_v.at[pl.ds(ch * CH, CH)]], y0_v, sem).wait()
            pltpu.async_copy(y_hbm.at[i1_v.at[pl.ds(ch * CH, CH)]], y1_v, sem).wait()

            def body(t, _):
                for j in range(D // 16):
                    sl = pl.ds(j * 16, 16)
                    y0_v[t, sl] = y0_v[t, sl] + y1_v[t, sl]
                return 0

            lax.fori_loop(0, CH, body, 0)
            pltpu.sync_copy(y0_v, out_hbm.at[pl.ds(base + ch * CH, CH)])

    return k(y_sorted, p0, p1)


def kernel(hidden_states, routing_weights, selected_experts, gate_proj, up_proj, down_proj):
    p, block_expert, nact = _routing_metadata(selected_experts)
    pk = p.reshape(T, K)
    p0 = pk[:, 0]
    p1 = pk[:, 1]
    rwb0 = jnp.broadcast_to(routing_weights[:, 0:1], (T, WCOL))
    rwb1 = jnp.broadcast_to(routing_weights[:, 1:2], (T, WCOL))

    x_sorted, w_sorted = _sc_dispatch(hidden_states, p0, p1, rwb0, rwb1)

    y_sorted = _grouped_ffn(block_expert, nact, x_sorted, gate_proj,
                            up_proj, down_proj, w_sorted)

    return _sc_combine(y_sorted, p0, p1)
